# baseline (speedup 1.0000x reference)
import jax
import jax.numpy as jnp
from jax import lax
from jax.experimental import pallas as pl
from jax.experimental.pallas import tpu as pltpu
from jax.experimental.pallas import tpu_sc as plsc

N = 100000
D = 128
G = 512
GD = 64
H = 128

NC = 2
NS = 16
NW = NC * NS
CHUNK = 128

FULL_CHUNKS = N // CHUNK
TAIL = N - FULL_CHUNKS * CHUNK
BASE_CH = FULL_CHUNKS // NW
EXTRA_W = FULL_CHUNKS - BASE_CH * NW
ROWS_PER_TILE = G // NS

CNT_BLK = 10000
CNT_NB = N // CNT_BLK
CNT_HI = 16
CNT_LO = 32


def _sc_body(xfeat_hbm, batch_hbm, sums_hbm,
             idx0_v, idx1_v, row0_v, row1_v, idx_t, row_t, tmp_v,
             sem0, sem1, acc_sh):
    cid = lax.axis_index("c")
    sid = lax.axis_index("s")
    wid = cid * NS + sid
    idx_b = (idx0_v, idx1_v)
    row_b = (row0_v, row1_v)
    sem_b = (sem0, sem1)

    for i in range(ROWS_PER_TILE):
        for j in range(D // 16):
            tmp_v[i, pl.ds(j * 16, 16)] = jnp.zeros((16,), jnp.float32)
    pltpu.sync_copy(tmp_v, acc_sh.at[pl.ds(sid * ROWS_PER_TILE, ROWS_PER_TILE), :])
    plsc.subcore_barrier()

    extra = wid < EXTRA_W
    ch0 = jnp.where(extra, wid * (BASE_CH + 1),
                    EXTRA_W * (BASE_CH + 1) + (wid - EXTRA_W) * BASE_CH)
    rbase = pl.multiple_of(ch0 * CHUNK, CHUNK)

    def start(b, base):
        base = pl.multiple_of(base, CHUNK)
        pltpu.async_copy(batch_hbm.at[pl.ds(base, CHUNK)], idx_b[b], sem_b[b])
        pltpu.async_copy(xfeat_hbm.at[pl.ds(base, CHUNK), :], row_b[b], sem_b[b])

    def wait(b):
        pltpu.make_async_copy(batch_hbm.at[pl.ds(0, CHUNK)], idx_b[b], sem_b[b]).wait()
        pltpu.make_async_copy(xfeat_hbm.at[pl.ds(0, CHUNK), :], row_b[b], sem_b[b]).wait()

    def scatter(b):
        pltpu.sync_copy(row_b[b], acc_sh.at[idx_b[b]], add=True)

    start(0, rbase)

    @pl.loop(0, BASE_CH // 2)
    def _(k):
        for b in (0, 1):
            c = 2 * k + b
            wait(b)
            nxt = c + 1

            @pl.when((nxt < BASE_CH) | (extra & (nxt == BASE_CH)))
            def _():
                start(1 - b, rbase + nxt * CHUNK)

            scatter(b)

    @pl.when(extra)
    def _():
        wait(0)
        scatter(0)

    @pl.when(wid == NW - 1)
    def _():
        tbase = FULL_CHUNKS * CHUNK
        pltpu.sync_copy(batch_hbm.at[pl.ds(tbase, TAIL)], idx_t)
        pltpu.sync_copy(xfeat_hbm.at[pl.ds(tbase, TAIL), :], row_t)
        pltpu.sync_copy(row_t, acc_sh.at[idx_t], add=True)

    plsc.subcore_barrier()

    r0 = sid * ROWS_PER_TILE
    pltpu.sync_copy(acc_sh.at[pl.ds(r0, ROWS_PER_TILE), :], tmp_v)
    pltpu.sync_copy(tmp_v, sums_hbm.at[cid, pl.ds(r0, ROWS_PER_TILE), :])


_sc_segsum = pl.kernel(
    _sc_body,
    out_type=jax.ShapeDtypeStruct((NC, G, D), jnp.float32),
    mesh=plsc.VectorSubcoreMesh(core_axis_name="c", subcore_axis_name="s",
                                num_cores=NC, num_subcores=NS),
    scratch_types=(
        pltpu.VMEM((CHUNK,), jnp.int32),
        pltpu.VMEM((CHUNK,), jnp.int32),
        pltpu.VMEM((CHUNK, D), jnp.float32),
        pltpu.VMEM((CHUNK, D), jnp.float32),
        pltpu.VMEM((TAIL,), jnp.int32),
        pltpu.VMEM((TAIL, D), jnp.float32),
        pltpu.VMEM((ROWS_PER_TILE, D), jnp.float32),
        pltpu.SemaphoreType.DMA,
        pltpu.SemaphoreType.DMA,
        pltpu.VMEM_SHARED((G, D), jnp.float32),
    ),
)


def _cnt_body(batch_ref, out_ref):
    i = pl.program_id(0)

    @pl.when(i == 0)
    def _():
        out_ref[...] = jnp.zeros_like(out_ref)

    b = batch_ref[0]
    hi = b >> 5
    lo = b & 31
    hi_oh = (hi == lax.broadcasted_iota(jnp.int32, (CNT_BLK, CNT_HI), 1)
             ).astype(jnp.float32)
    lo_oh = (lo == lax.broadcasted_iota(jnp.int32, (CNT_BLK, CNT_LO), 1)
             ).astype(jnp.float32)
    out_ref[...] += lax.dot_general(
        hi_oh, lo_oh, dimension_numbers=(((0,), (0,)), ((), ())),
        preferred_element_type=jnp.float32)


_tc_counts = pl.pallas_call(
    _cnt_body,
    grid=(CNT_NB,),
    in_specs=[pl.BlockSpec((1, CNT_BLK, 1), lambda i: (i, 0, 0))],
    out_specs=pl.BlockSpec((CNT_HI, CNT_LO), lambda i: (0, 0)),
    out_shape=jax.ShapeDtypeStruct((CNT_HI, CNT_LO), jnp.float32),
)


def _tc_body(sums_ref, cnt_ref, u_ref, w1_ref, b1_ref, w2_ref, b2_ref, out_ref):
    s = sums_ref[0] + sums_ref[1]
    mean = s / jnp.maximum(cnt_ref[...], 1.0)
    x = (jnp.dot(u_ref[...], w1_ref[:GD], preferred_element_type=jnp.float32)
         + jnp.dot(mean, w1_ref[GD:], preferred_element_type=jnp.float32)
         + b1_ref[...])
    h = jnp.maximum(x, 0.0)
    out_ref[...] = (jnp.dot(h, w2_ref[...], preferred_element_type=jnp.float32)
                    + b2_ref[...])


_tc_mlp = pl.pallas_call(
    _tc_body,
    out_shape=jax.ShapeDtypeStruct((G, GD), jnp.float32),
)


def kernel(xfeat, T, edge_index, edge_attr, u, batch, W1, b1, W2, b2):
    sums = _sc_segsum(xfeat, batch)
    cnt = _tc_counts(batch.reshape(CNT_NB, CNT_BLK, 1)).reshape(G, 1)
    return _tc_mlp(sums, cnt, u, W1, b1.reshape(1, H), W2, b2.reshape(1, GD))


# speedup vs baseline: 2.2431x; 2.2431x over previous
import jax
import jax.numpy as jnp
from jax import lax
from jax.experimental import pallas as pl
from jax.experimental.pallas import tpu as pltpu
from jax.experimental.pallas import tpu_sc as plsc

N = 100000
D = 128
G = 512
GD = 64
H = 128

NC = 2
NS = 16
NW = NC * NS
CHUNK = 128

FULL_CHUNKS = N // CHUNK
TAIL = N - FULL_CHUNKS * CHUNK
BASE_CH = FULL_CHUNKS // NW
EXTRA_W = FULL_CHUNKS - BASE_CH * NW
ROWS_PER_TILE = G // NS

CNT_BLK = 10000
CNT_NB = N // CNT_BLK
CNT_HI = 16
CNT_LO = 32


def _sc_body(xfeat_hbm, batch_hbm, sums_hbm,
             idx0_v, idx1_v, row0_v, row1_v, idx_t, row_t, tmp_v,
             sem0, sem1, acc_sh):
    cid = lax.axis_index("c")
    sid = lax.axis_index("s")
    wid = cid * NS + sid
    idx_b = (idx0_v, idx1_v)
    row_b = (row0_v, row1_v)
    sem_b = (sem0, sem1)

    for i in range(ROWS_PER_TILE):
        for j in range(D // 16):
            tmp_v[i, pl.ds(j * 16, 16)] = jnp.zeros((16,), jnp.float32)
    pltpu.sync_copy(tmp_v, acc_sh.at[pl.ds(sid * ROWS_PER_TILE, ROWS_PER_TILE), :])
    plsc.subcore_barrier()

    extra = wid < EXTRA_W
    ch0 = jnp.where(extra, wid * (BASE_CH + 1),
                    EXTRA_W * (BASE_CH + 1) + (wid - EXTRA_W) * BASE_CH)
    rbase = pl.multiple_of(ch0 * CHUNK, CHUNK)

    def start(b, base):
        base = pl.multiple_of(base, CHUNK)
        pltpu.async_copy(batch_hbm.at[pl.ds(base, CHUNK)], idx_b[b], sem_b[b])
        pltpu.async_copy(xfeat_hbm.at[pl.ds(base, CHUNK), :], row_b[b], sem_b[b])

    def wait(b):
        pltpu.make_async_copy(batch_hbm.at[pl.ds(0, CHUNK)], idx_b[b], sem_b[b]).wait()
        pltpu.make_async_copy(xfeat_hbm.at[pl.ds(0, CHUNK), :], row_b[b], sem_b[b]).wait()

    def scatter(b):
        pltpu.sync_copy(row_b[b], acc_sh.at[idx_b[b]], add=True)

    start(0, rbase)

    @pl.loop(0, BASE_CH // 2)
    def _(k):
        for b in (0, 1):
            c = 2 * k + b
            wait(b)
            nxt = c + 1

            @pl.when((nxt < BASE_CH) | (extra & (nxt == BASE_CH)))
            def _():
                start(1 - b, rbase + nxt * CHUNK)

            scatter(b)

    @pl.when(extra)
    def _():
        wait(0)
        scatter(0)

    @pl.when(wid == NW - 1)
    def _():
        tbase = FULL_CHUNKS * CHUNK
        pltpu.sync_copy(batch_hbm.at[pl.ds(tbase, TAIL)], idx_t)
        pltpu.sync_copy(xfeat_hbm.at[pl.ds(tbase, TAIL), :], row_t)
        pltpu.sync_copy(row_t, acc_sh.at[idx_t], add=True)

    plsc.subcore_barrier()

    r0 = sid * ROWS_PER_TILE
    pltpu.sync_copy(acc_sh.at[pl.ds(r0, ROWS_PER_TILE), :], tmp_v)
    pltpu.sync_copy(tmp_v, sums_hbm.at[cid, pl.ds(r0, ROWS_PER_TILE), :])


_sc_segsum = pl.kernel(
    _sc_body,
    out_type=jax.ShapeDtypeStruct((NC, G, D), jnp.float32),
    mesh=plsc.VectorSubcoreMesh(core_axis_name="c", subcore_axis_name="s",
                                num_cores=NC, num_subcores=NS),
    scratch_types=(
        pltpu.VMEM((CHUNK,), jnp.int32),
        pltpu.VMEM((CHUNK,), jnp.int32),
        pltpu.VMEM((CHUNK, D), jnp.float32),
        pltpu.VMEM((CHUNK, D), jnp.float32),
        pltpu.VMEM((TAIL,), jnp.int32),
        pltpu.VMEM((TAIL, D), jnp.float32),
        pltpu.VMEM((ROWS_PER_TILE, D), jnp.float32),
        pltpu.SemaphoreType.DMA,
        pltpu.SemaphoreType.DMA,
        pltpu.VMEM_SHARED((G, D), jnp.float32),
    ),
)


def _cnt_body(batch_ref, out_ref):
    i = pl.program_id(0)

    @pl.when(i == 0)
    def _():
        out_ref[...] = jnp.zeros_like(out_ref)

    b = batch_ref[0]
    hi = b >> 5
    lo = b & 31
    hi_oh = (hi == lax.broadcasted_iota(jnp.int32, (CNT_HI, CNT_BLK), 0)
             ).astype(jnp.float32)
    lo_oh = (lo == lax.broadcasted_iota(jnp.int32, (CNT_LO, CNT_BLK), 0)
             ).astype(jnp.float32)
    out_ref[...] += lax.dot_general(
        hi_oh, lo_oh, dimension_numbers=(((1,), (1,)), ((), ())),
        preferred_element_type=jnp.float32)


_tc_counts = pl.pallas_call(
    _cnt_body,
    grid=(CNT_NB,),
    in_specs=[pl.BlockSpec((1, 1, CNT_BLK), lambda i: (i, 0, 0))],
    out_specs=pl.BlockSpec((CNT_HI, CNT_LO), lambda i: (0, 0)),
    out_shape=jax.ShapeDtypeStruct((CNT_HI, CNT_LO), jnp.float32),
)


def _tc_body(sums_ref, cnt_ref, u_ref, w1_ref, b1_ref, w2_ref, b2_ref, out_ref):
    s = sums_ref[0] + sums_ref[1]
    mean = s / jnp.maximum(cnt_ref[...], 1.0)
    x = (jnp.dot(u_ref[...], w1_ref[:GD], preferred_element_type=jnp.float32)
         + jnp.dot(mean, w1_ref[GD:], preferred_element_type=jnp.float32)
         + b1_ref[...])
    h = jnp.maximum(x, 0.0)
    out_ref[...] = (jnp.dot(h, w2_ref[...], preferred_element_type=jnp.float32)
                    + b2_ref[...])


_tc_mlp = pl.pallas_call(
    _tc_body,
    out_shape=jax.ShapeDtypeStruct((G, GD), jnp.float32),
)


def kernel(xfeat, T, edge_index, edge_attr, u, batch, W1, b1, W2, b2):
    sums = _sc_segsum(xfeat, batch)
    cnt = _tc_counts(batch.reshape(CNT_NB, 1, CNT_BLK)).reshape(G, 1)
    return _tc_mlp(sums, cnt, u, W1, b1.reshape(1, H), W2, b2.reshape(1, GD))


# speedup vs baseline: 2.4140x; 1.0762x over previous
import jax
import jax.numpy as jnp
from jax import lax
from jax.experimental import pallas as pl
from jax.experimental.pallas import tpu as pltpu
from jax.experimental.pallas import tpu_sc as plsc

N = 100000
D = 128
G = 512
GD = 64
H = 128

NC = 2
NS = 16
NW = NC * NS
CHUNK = 128

SC_CH = 12
TC_BLK = 3176
TC_NB = 16
TC_ROWS = TC_BLK * TC_NB
TC_CHUNKS = TC_ROWS // CHUNK
SC_ROWS = NW * SC_CH * CHUNK
TAIL = N - TC_ROWS - SC_ROWS
ROWS_PER_TILE = G // NS

CNT_BLK = 10000
CNT_NB = N // CNT_BLK
CNT_HI = 16
CNT_LO = 32


def _sc_body(xfeat_hbm, batch_hbm, sums_hbm,
             idx0_v, idx1_v, row0_v, row1_v, idx_t, row_t, tmp_v,
             sem0, sem1, acc_sh):
    cid = lax.axis_index("c")
    sid = lax.axis_index("s")
    wid = cid * NS + sid
    idx_b = (idx0_v, idx1_v)
    row_b = (row0_v, row1_v)
    sem_b = (sem0, sem1)

    for i in range(ROWS_PER_TILE):
        for j in range(D // 16):
            tmp_v[i, pl.ds(j * 16, 16)] = jnp.zeros((16,), jnp.float32)
    pltpu.sync_copy(tmp_v, acc_sh.at[pl.ds(sid * ROWS_PER_TILE, ROWS_PER_TILE), :])
    plsc.subcore_barrier()

    rbase = pl.multiple_of((TC_CHUNKS + wid * SC_CH) * CHUNK, CHUNK)

    def start(b, base):
        base = pl.multiple_of(base, CHUNK)
        pltpu.async_copy(batch_hbm.at[pl.ds(base, CHUNK)], idx_b[b], sem_b[b])
        pltpu.async_copy(xfeat_hbm.at[pl.ds(base, CHUNK), :], row_b[b], sem_b[b])

    def wait(b):
        pltpu.make_async_copy(batch_hbm.at[pl.ds(0, CHUNK)], idx_b[b], sem_b[b]).wait()
        pltpu.make_async_copy(xfeat_hbm.at[pl.ds(0, CHUNK), :], row_b[b], sem_b[b]).wait()

    def scatter(b):
        pltpu.sync_copy(row_b[b], acc_sh.at[idx_b[b]], add=True)

    start(0, rbase)

    @pl.loop(0, SC_CH // 2)
    def _(k):
        for b in (0, 1):
            c = 2 * k + b
            wait(b)

            @pl.when(c + 1 < SC_CH)
            def _():
                start(1 - b, rbase + (c + 1) * CHUNK)

            scatter(b)

    @pl.when(wid == NW - 1)
    def _():
        tbase = TC_ROWS + SC_ROWS
        pltpu.sync_copy(batch_hbm.at[pl.ds(tbase, TAIL)], idx_t)
        pltpu.sync_copy(xfeat_hbm.at[pl.ds(tbase, TAIL), :], row_t)
        pltpu.sync_copy(row_t, acc_sh.at[idx_t], add=True)

    plsc.subcore_barrier()

    r0 = sid * ROWS_PER_TILE
    pltpu.sync_copy(acc_sh.at[pl.ds(r0, ROWS_PER_TILE), :], tmp_v)
    pltpu.sync_copy(tmp_v, sums_hbm.at[cid, pl.ds(r0, ROWS_PER_TILE), :])


_sc_segsum = pl.kernel(
    _sc_body,
    out_type=jax.ShapeDtypeStruct((NC, G, D), jnp.float32),
    mesh=plsc.VectorSubcoreMesh(core_axis_name="c", subcore_axis_name="s",
                                num_cores=NC, num_subcores=NS),
    scratch_types=(
        pltpu.VMEM((CHUNK,), jnp.int32),
        pltpu.VMEM((CHUNK,), jnp.int32),
        pltpu.VMEM((CHUNK, D), jnp.float32),
        pltpu.VMEM((CHUNK, D), jnp.float32),
        pltpu.VMEM((TAIL,), jnp.int32),
        pltpu.VMEM((TAIL, D), jnp.float32),
        pltpu.VMEM((ROWS_PER_TILE, D), jnp.float32),
        pltpu.SemaphoreType.DMA,
        pltpu.SemaphoreType.DMA,
        pltpu.VMEM_SHARED((G, D), jnp.float32),
    ),
)


def _tcsum_body(x_ref, batch_ref, out_ref):
    i = pl.program_id(0)

    @pl.when(i == 0)
    def _():
        out_ref[...] = jnp.zeros_like(out_ref)

    b = batch_ref[0]
    oh = (b == lax.broadcasted_iota(jnp.int32, (G, TC_BLK), 0)
          ).astype(jnp.float32)
    out_ref[...] += lax.dot_general(
        oh, x_ref[...], dimension_numbers=(((1,), (0,)), ((), ())),
        preferred_element_type=jnp.float32)


_tc_segsum = pl.pallas_call(
    _tcsum_body,
    grid=(TC_NB,),
    in_specs=[
        pl.BlockSpec((TC_BLK, D), lambda i: (i, 0)),
        pl.BlockSpec((1, 1, TC_BLK), lambda i: (i, 0, 0)),
    ],
    out_specs=pl.BlockSpec((G, D), lambda i: (0, 0)),
    out_shape=jax.ShapeDtypeStruct((G, D), jnp.float32),
)


def _cnt_body(batch_ref, out_ref):
    i = pl.program_id(0)

    @pl.when(i == 0)
    def _():
        out_ref[...] = jnp.zeros_like(out_ref)

    b = batch_ref[0]
    hi = b >> 5
    lo = b & 31
    hi_oh = (hi == lax.broadcasted_iota(jnp.int32, (CNT_HI, CNT_BLK), 0)
             ).astype(jnp.float32)
    lo_oh = (lo == lax.broadcasted_iota(jnp.int32, (CNT_LO, CNT_BLK), 0)
             ).astype(jnp.float32)
    out_ref[...] += lax.dot_general(
        hi_oh, lo_oh, dimension_numbers=(((1,), (1,)), ((), ())),
        preferred_element_type=jnp.float32)


_tc_counts = pl.pallas_call(
    _cnt_body,
    grid=(CNT_NB,),
    in_specs=[pl.BlockSpec((1, 1, CNT_BLK), lambda i: (i, 0, 0))],
    out_specs=pl.BlockSpec((CNT_HI, CNT_LO), lambda i: (0, 0)),
    out_shape=jax.ShapeDtypeStruct((CNT_HI, CNT_LO), jnp.float32),
)


def _tc_body(sums_ref, tcsum_ref, cnt_ref, u_ref, w1_ref, b1_ref, w2_ref,
             b2_ref, out_ref):
    s = sums_ref[0] + sums_ref[1] + tcsum_ref[...]
    mean = s / jnp.maximum(cnt_ref[...], 1.0)
    x = (jnp.dot(u_ref[...], w1_ref[:GD], preferred_element_type=jnp.float32)
         + jnp.dot(mean, w1_ref[GD:], preferred_element_type=jnp.float32)
         + b1_ref[...])
    h = jnp.maximum(x, 0.0)
    out_ref[...] = (jnp.dot(h, w2_ref[...], preferred_element_type=jnp.float32)
                    + b2_ref[...])


_tc_mlp = pl.pallas_call(
    _tc_body,
    out_shape=jax.ShapeDtypeStruct((G, GD), jnp.float32),
)


def kernel(xfeat, T, edge_index, edge_attr, u, batch, W1, b1, W2, b2):
    sums = _sc_segsum(xfeat, batch)
    tcsum = _tc_segsum(xfeat, batch[:TC_ROWS].reshape(TC_NB, 1, TC_BLK))
    cnt = _tc_counts(batch.reshape(CNT_NB, 1, CNT_BLK)).reshape(G, 1)
    return _tc_mlp(sums, tcsum, cnt, u, W1, b1.reshape(1, H), W2,
                   b2.reshape(1, GD))


# speedup vs baseline: 2.8317x; 1.1730x over previous
import jax
import jax.numpy as jnp
from jax import lax
from jax.experimental import pallas as pl
from jax.experimental.pallas import tpu as pltpu
from jax.experimental.pallas import tpu_sc as plsc

N = 100000
D = 128
G = 512
GD = 64
H = 128

NC = 2
NS = 16
NW = NC * NS
CHUNK = 128

SC_CH = 14
TC_BLK = 5328
TC_NB = 8
TC_ROWS = TC_BLK * TC_NB
TC_CHUNKS = TC_ROWS // CHUNK
SC_ROWS = NW * SC_CH * CHUNK
TAIL = N - TC_ROWS - SC_ROWS
ROWS_PER_TILE = G // NS

CNT_BLK = 25000
CNT_NB = N // CNT_BLK
CNT_HI = 16
CNT_LO = 32


def _sc_body(xfeat_hbm, batch_hbm, sums_hbm,
             idx0_v, idx1_v, row0_v, row1_v, idx_t, row_t, tmp_v,
             sem0, sem1, acc_sh):
    cid = lax.axis_index("c")
    sid = lax.axis_index("s")
    wid = cid * NS + sid
    idx_b = (idx0_v, idx1_v)
    row_b = (row0_v, row1_v)
    sem_b = (sem0, sem1)

    for i in range(ROWS_PER_TILE):
        for j in range(D // 16):
            tmp_v[i, pl.ds(j * 16, 16)] = jnp.zeros((16,), jnp.float32)
    pltpu.sync_copy(tmp_v, acc_sh.at[pl.ds(sid * ROWS_PER_TILE, ROWS_PER_TILE), :])
    plsc.subcore_barrier()

    rbase = pl.multiple_of((TC_CHUNKS + wid * SC_CH) * CHUNK, CHUNK)

    def start(b, base):
        base = pl.multiple_of(base, CHUNK)
        pltpu.async_copy(batch_hbm.at[pl.ds(base, CHUNK)], idx_b[b], sem_b[b])
        pltpu.async_copy(xfeat_hbm.at[pl.ds(base, CHUNK), :], row_b[b], sem_b[b])

    def wait(b):
        pltpu.make_async_copy(batch_hbm.at[pl.ds(0, CHUNK)], idx_b[b], sem_b[b]).wait()
        pltpu.make_async_copy(xfeat_hbm.at[pl.ds(0, CHUNK), :], row_b[b], sem_b[b]).wait()

    def scatter(b):
        pltpu.sync_copy(row_b[b], acc_sh.at[idx_b[b]], add=True)

    start(0, rbase)

    @pl.loop(0, SC_CH // 2)
    def _(k):
        for b in (0, 1):
            c = 2 * k + b
            wait(b)

            @pl.when(c + 1 < SC_CH)
            def _():
                start(1 - b, rbase + (c + 1) * CHUNK)

            scatter(b)

    @pl.when(wid == NW - 1)
    def _():
        tbase = TC_ROWS + SC_ROWS
        pltpu.sync_copy(batch_hbm.at[pl.ds(tbase, TAIL)], idx_t)
        pltpu.sync_copy(xfeat_hbm.at[pl.ds(tbase, TAIL), :], row_t)
        pltpu.sync_copy(row_t, acc_sh.at[idx_t], add=True)

    plsc.subcore_barrier()

    r0 = sid * ROWS_PER_TILE
    pltpu.sync_copy(acc_sh.at[pl.ds(r0, ROWS_PER_TILE), :], tmp_v)
    pltpu.sync_copy(tmp_v, sums_hbm.at[cid, pl.ds(r0, ROWS_PER_TILE), :])


_sc_segsum = pl.kernel(
    _sc_body,
    out_type=jax.ShapeDtypeStruct((NC, G, D), jnp.float32),
    mesh=plsc.VectorSubcoreMesh(core_axis_name="c", subcore_axis_name="s",
                                num_cores=NC, num_subcores=NS),
    scratch_types=(
        pltpu.VMEM((CHUNK,), jnp.int32),
        pltpu.VMEM((CHUNK,), jnp.int32),
        pltpu.VMEM((CHUNK, D), jnp.float32),
        pltpu.VMEM((CHUNK, D), jnp.float32),
        pltpu.VMEM((TAIL,), jnp.int32),
        pltpu.VMEM((TAIL, D), jnp.float32),
        pltpu.VMEM((ROWS_PER_TILE, D), jnp.float32),
        pltpu.SemaphoreType.DMA,
        pltpu.SemaphoreType.DMA,
        pltpu.VMEM_SHARED((G, D), jnp.float32),
    ),
)


def _tcsum_body(x_ref, batch_ref, out_ref):
    i = pl.program_id(0)

    @pl.when(i == 0)
    def _():
        out_ref[...] = jnp.zeros_like(out_ref)

    b = batch_ref[0]
    oh = (b == lax.broadcasted_iota(jnp.int32, (G, TC_BLK), 0)
          ).astype(jnp.float32)
    out_ref[...] += lax.dot_general(
        oh, x_ref[...], dimension_numbers=(((1,), (0,)), ((), ())),
        preferred_element_type=jnp.float32)


_tc_segsum = pl.pallas_call(
    _tcsum_body,
    grid=(TC_NB,),
    in_specs=[
        pl.BlockSpec((TC_BLK, D), lambda i: (i, 0)),
        pl.BlockSpec((1, 1, TC_BLK), lambda i: (i, 0, 0)),
    ],
    out_specs=pl.BlockSpec((G, D), lambda i: (0, 0)),
    out_shape=jax.ShapeDtypeStruct((G, D), jnp.float32),
)


def _cnt_body(batch_ref, out_ref):
    i = pl.program_id(0)

    @pl.when(i == 0)
    def _():
        out_ref[...] = jnp.zeros_like(out_ref)

    b = batch_ref[0]
    hi = b >> 5
    lo = b & 31
    hi_oh = (hi == lax.broadcasted_iota(jnp.int32, (CNT_HI, CNT_BLK), 0)
             ).astype(jnp.float32)
    lo_oh = (lo == lax.broadcasted_iota(jnp.int32, (CNT_LO, CNT_BLK), 0)
             ).astype(jnp.float32)
    out_ref[...] += lax.dot_general(
        hi_oh, lo_oh, dimension_numbers=(((1,), (1,)), ((), ())),
        preferred_element_type=jnp.float32)


_tc_counts = pl.pallas_call(
    _cnt_body,
    grid=(CNT_NB,),
    in_specs=[pl.BlockSpec((1, 1, CNT_BLK), lambda i: (i, 0, 0))],
    out_specs=pl.BlockSpec((CNT_HI, CNT_LO), lambda i: (0, 0)),
    out_shape=jax.ShapeDtypeStruct((CNT_HI, CNT_LO), jnp.float32),
)


def _tc_body(sums_ref, tcsum_ref, cnt_ref, u_ref, w1_ref, b1_ref, w2_ref,
             b2_ref, out_ref):
    s = sums_ref[0] + sums_ref[1] + tcsum_ref[...]
    g_hi = lax.broadcasted_iota(jnp.int32, (G, CNT_HI), 0) >> 5
    g_lo = lax.broadcasted_iota(jnp.int32, (G, CNT_LO), 0) & 31
    ahi = (g_hi == lax.broadcasted_iota(jnp.int32, (G, CNT_HI), 1)
           ).astype(jnp.float32)
    alo = (g_lo == lax.broadcasted_iota(jnp.int32, (G, CNT_LO), 1)
           ).astype(jnp.float32)
    t = jnp.dot(ahi, cnt_ref[...], preferred_element_type=jnp.float32)
    cnt = jnp.sum(t * alo, axis=1, keepdims=True)
    mean = s / jnp.maximum(cnt, 1.0)
    x = (jnp.dot(u_ref[...], w1_ref[:GD], preferred_element_type=jnp.float32)
         + jnp.dot(mean, w1_ref[GD:], preferred_element_type=jnp.float32)
         + b1_ref[...])
    h = jnp.maximum(x, 0.0)
    out_ref[...] = (jnp.dot(h, w2_ref[...], preferred_element_type=jnp.float32)
                    + b2_ref[...])


_tc_mlp = pl.pallas_call(
    _tc_body,
    out_shape=jax.ShapeDtypeStruct((G, GD), jnp.float32),
)


def kernel(xfeat, T, edge_index, edge_attr, u, batch, W1, b1, W2, b2):
    sums = _sc_segsum(xfeat, batch)
    tcsum = _tc_segsum(xfeat, batch[:TC_ROWS].reshape(TC_NB, 1, TC_BLK))
    cnt = _tc_counts(batch.reshape(CNT_NB, 1, CNT_BLK))
    return _tc_mlp(sums, tcsum, cnt, u, W1, b1.reshape(1, H), W2,
                   b2.reshape(1, GD))
